# baseline (speedup 1.0000x reference)
import functools

import jax
import jax.numpy as jnp
from jax import lax
from jax.experimental import pallas as pl
from jax.experimental.pallas import tpu as pltpu
from jax.experimental.pallas import tpu_sc as plsc

N = 10000
E = 320000
F = 128
H = 256
NC = 2
NS = 16
WIN = 128
W_AGG = 160
EPP = W_AGG * WIN
EPAD = EPP * NS
W_DEG = 80
DPP = W_DEG * WIN
PAD_ROWS = 16
NPT = N + PAD_ROWS
ROW_CHUNK = 624
ROW_LAST = N - 15 * ROW_CHUNK
TROW_LAST = NPT - 15 * ROW_CHUNK
RB = 2000

_mesh = plsc.VectorSubcoreMesh(core_axis_name="c", subcore_axis_name="s")


def _chunked_copy(copy_fn, sid, last_size):
    base = sid * ROW_CHUNK

    @pl.when(sid < 15)
    def _():
        copy_fn(base, ROW_CHUNK)

    @pl.when(sid == 15)
    def _():
        copy_fn(base, last_size)


def _deg_body(dstp_hbm, zeros_hbm, degA_hbm, degB_hbm, idx_v, ones_v, zsh):
    cid = lax.axis_index("c")
    sid = lax.axis_index("s")
    wid = cid * NS + sid

    ones = jnp.full((16,), 1.0, dtype=jnp.float32)

    @pl.loop(0, WIN)
    def _(i):
        @pl.loop(0, F, step=16)
        def _(j):
            ones_v[i, pl.ds(j, 16)] = ones

    def init(base, size):
        pltpu.sync_copy(zeros_hbm.at[pl.ds(0, size)],
                        zsh.at[pl.ds(base, size)])

    _chunked_copy(init, sid, TROW_LAST)
    pltpu.sync_copy(dstp_hbm.at[pl.ds(wid * W_DEG, W_DEG)], idx_v)
    plsc.subcore_barrier()

    @pl.loop(0, W_DEG)
    def _(w):
        pltpu.sync_copy(ones_v, zsh.at[idx_v.at[w]], add=True)

    plsc.subcore_barrier()

    def outA(base, size):
        pltpu.sync_copy(zsh.at[pl.ds(base, size)],
                        degA_hbm.at[pl.ds(base, size)])

    def outB(base, size):
        pltpu.sync_copy(zsh.at[pl.ds(base, size)],
                        degB_hbm.at[pl.ds(base, size)])

    @pl.when(cid == 0)
    def _():
        _chunked_copy(outA, sid, ROW_LAST)

    @pl.when(cid == 1)
    def _():
        _chunked_copy(outB, sid, ROW_LAST)


_deg_call = functools.partial(
    pl.kernel,
    out_type=[jax.ShapeDtypeStruct((N, F), jnp.float32),
              jax.ShapeDtypeStruct((N, F), jnp.float32)],
    mesh=_mesh,
    scratch_types=[
        pltpu.VMEM((W_DEG, WIN), jnp.int32),
        pltpu.VMEM((WIN, F), jnp.float32),
        pltpu.VMEM_SHARED((NPT, F), jnp.float32),
    ],
)(_deg_body)


AWIN = 64
W_SUB = EPP // AWIN
CH = 16
NCH = W_SUB // CH
NSLOT = 4


def _agg_body(y0_hbm, y1_hbm, srcp_hbm, dstp_hbm, z0_hbm, z1_hbm,
              cs0, cs1, cd0, cd1, rows0, rows1, rows2, rows3,
              semc0, semc1, semg0, semg1, semg2, semg3,
              sems0, sems1, sems2, sems3, zsh):
    cid = lax.axis_index("c")
    sid = lax.axis_index("s")
    cs = (cs0, cs1)
    cd = (cd0, cd1)
    semc = (semc0, semc1)
    rows = (rows0, rows1, rows2, rows3)
    semg = (semg0, semg1, semg2, semg3)
    sems = (sems0, sems1, sems2, sems3)

    def run(y_hbm, z_hbm):
        def seed(base, size):
            pltpu.sync_copy(y_hbm.at[pl.ds(base, size)],
                            zsh.at[pl.ds(base, size)])

        _chunked_copy(seed, sid, ROW_LAST)
        plsc.subcore_barrier()

        def start_chunk(c, s):
            base = sid * W_SUB + c * CH
            pltpu.async_copy(srcp_hbm.at[pl.ds(base, CH)], cs[s], semc[s])
            pltpu.async_copy(dstp_hbm.at[pl.ds(base, CH)], cd[s], semc[s])

        def wait_chunk(s):
            pltpu.make_async_copy(srcp_hbm.at[pl.ds(0, CH)], cs[s],
                                  semc[s]).wait()
            pltpu.make_async_copy(dstp_hbm.at[pl.ds(0, CH)], cd[s],
                                  semc[s]).wait()

        def start_gather(s, k, b):
            pltpu.async_copy(y_hbm.at[cs[s].at[k]], rows[b], semg[b])

        def wait_gather(b):
            pltpu.make_async_copy(y_hbm.at[cs[0].at[0]], rows[b],
                                  semg[b]).wait()

        def start_scatter(s, k, b):
            pltpu.async_copy(rows[b], zsh.at[cd[s].at[k]], sems[b],
                             add=True)

        def wait_scatter(b):
            pltpu.make_async_copy(rows[b], zsh.at[cd[0].at[0]],
                                  sems[b]).wait()

        start_chunk(0, 0)
        start_chunk(1, 1)

        @pl.loop(0, NCH, step=2)
        def _(c):
            for s in range(2):
                wait_chunk(s)
                start_gather(s, 0, 0)
                start_gather(s, 1, 1)
                for k in range(CH):
                    b = k % NSLOT
                    bn = (k + 2) % NSLOT
                    wait_gather(b)
                    start_scatter(s, k, b)
                    if k + 2 < CH:
                        if k >= 2:
                            wait_scatter(bn)
                        start_gather(s, k + 2, bn)
                for b in range(NSLOT):
                    wait_scatter(b)

                @pl.when(c + 2 + s < NCH)
                def _():
                    start_chunk(c + 2 + s, s)

        plsc.subcore_barrier()

        def out(base, size):
            pltpu.sync_copy(zsh.at[pl.ds(base, size)],
                            z_hbm.at[pl.ds(base, size)])

        _chunked_copy(out, sid, ROW_LAST)

    @pl.when(cid == 0)
    def _():
        run(y0_hbm, z0_hbm)

    @pl.when(cid == 1)
    def _():
        run(y1_hbm, z1_hbm)


_agg_call = functools.partial(
    pl.kernel,
    out_type=[jax.ShapeDtypeStruct((N, F), jnp.float32),
              jax.ShapeDtypeStruct((N, F), jnp.float32)],
    mesh=_mesh,
    scratch_types=[
        pltpu.VMEM((CH, AWIN), jnp.int32),
        pltpu.VMEM((CH, AWIN), jnp.int32),
        pltpu.VMEM((CH, AWIN), jnp.int32),
        pltpu.VMEM((CH, AWIN), jnp.int32),
        pltpu.VMEM((AWIN, F), jnp.float32),
        pltpu.VMEM((AWIN, F), jnp.float32),
        pltpu.VMEM((AWIN, F), jnp.float32),
        pltpu.VMEM((AWIN, F), jnp.float32),
        pltpu.SemaphoreType.DMA,
        pltpu.SemaphoreType.DMA,
        pltpu.SemaphoreType.DMA,
        pltpu.SemaphoreType.DMA,
        pltpu.SemaphoreType.DMA,
        pltpu.SemaphoreType.DMA,
        pltpu.SemaphoreType.DMA,
        pltpu.SemaphoreType.DMA,
        pltpu.SemaphoreType.DMA,
        pltpu.SemaphoreType.DMA,
        pltpu.VMEM_SHARED((NPT, F), jnp.float32),
    ],
)(_agg_body)


def _mm_body(x_ref, w_ref, o_ref):
    o_ref[...] = jnp.dot(x_ref[...], w_ref[...],
                         preferred_element_type=jnp.float32,
                         precision=lax.Precision.HIGHEST)


def _tc_matmul(x, w):
    m, k = x.shape
    _, n = w.shape
    return pl.pallas_call(
        _mm_body,
        grid=(m // RB,),
        in_specs=[pl.BlockSpec((RB, k), lambda i: (i, 0)),
                  pl.BlockSpec((k, n), lambda i: (0, 0))],
        out_specs=pl.BlockSpec((RB, n), lambda i: (i, 0)),
        out_shape=jax.ShapeDtypeStruct((m, n), jnp.float32),
    )(x, w)


def _scale_body(xw_ref, degA_ref, degB_ref, y0_ref, y1_ref):
    deg = degA_ref[:, 0:1] + degB_ref[:, 0:1]
    dinv = lax.rsqrt(deg + 1.0)
    y = xw_ref[...] * dinv
    y0_ref[...] = y[:, :F]
    y1_ref[...] = y[:, F:]


def _tc_scale_split(xw, degA, degB):
    return pl.pallas_call(
        _scale_body,
        grid=(N // RB,),
        in_specs=[pl.BlockSpec((RB, H), lambda i: (i, 0)),
                  pl.BlockSpec((RB, F), lambda i: (i, 0)),
                  pl.BlockSpec((RB, F), lambda i: (i, 0))],
        out_specs=[pl.BlockSpec((RB, F), lambda i: (i, 0)),
                   pl.BlockSpec((RB, F), lambda i: (i, 0))],
        out_shape=[jax.ShapeDtypeStruct((N, F), jnp.float32),
                   jax.ShapeDtypeStruct((N, F), jnp.float32)],
    )(xw, degA, degB)


def _mid_body(z0_ref, z1_ref, degA_ref, degB_ref, b_ref, w_ref,
              y0_ref, y1_ref):
    deg = degA_ref[:, 0:1] + degB_ref[:, 0:1]
    dinv = lax.rsqrt(deg + 1.0)
    h = jnp.concatenate([z0_ref[...], z1_ref[...]], axis=1)
    h = jnp.maximum(h * dinv + b_ref[...], 0.0)
    xw = jnp.dot(h, w_ref[...],
                 preferred_element_type=jnp.float32,
                 precision=lax.Precision.HIGHEST)
    y = xw * dinv
    y0_ref[...] = y[:, :F]
    y1_ref[...] = y[:, F:]


def _tc_mid(z0, z1, degA, degB, b, w):
    return pl.pallas_call(
        _mid_body,
        grid=(N // RB,),
        in_specs=[pl.BlockSpec((RB, F), lambda i: (i, 0)),
                  pl.BlockSpec((RB, F), lambda i: (i, 0)),
                  pl.BlockSpec((RB, F), lambda i: (i, 0)),
                  pl.BlockSpec((RB, F), lambda i: (i, 0)),
                  pl.BlockSpec((1, H), lambda i: (0, 0)),
                  pl.BlockSpec((H, H), lambda i: (0, 0))],
        out_specs=[pl.BlockSpec((RB, F), lambda i: (i, 0)),
                   pl.BlockSpec((RB, F), lambda i: (i, 0))],
        out_shape=[jax.ShapeDtypeStruct((N, F), jnp.float32),
                   jax.ShapeDtypeStruct((N, F), jnp.float32)],
    )(z0, z1, degA, degB, b, w)


def _fin_body(z0_ref, z1_ref, degA_ref, degB_ref, b_ref, o_ref):
    deg = degA_ref[:, 0:1] + degB_ref[:, 0:1]
    dinv = lax.rsqrt(deg + 1.0)
    h = jnp.concatenate([z0_ref[...], z1_ref[...]], axis=1)
    o_ref[...] = jnp.maximum(h * dinv + b_ref[...], 0.0)


def _tc_final(z0, z1, degA, degB, b):
    return pl.pallas_call(
        _fin_body,
        grid=(N // RB,),
        in_specs=[pl.BlockSpec((RB, F), lambda i: (i, 0)),
                  pl.BlockSpec((RB, F), lambda i: (i, 0)),
                  pl.BlockSpec((RB, F), lambda i: (i, 0)),
                  pl.BlockSpec((RB, F), lambda i: (i, 0)),
                  pl.BlockSpec((1, H), lambda i: (0, 0))],
        out_specs=pl.BlockSpec((RB, H), lambda i: (i, 0)),
        out_shape=jax.ShapeDtypeStruct((N, H), jnp.float32),
    )(z0, z1, degA, degB, b)


def kernel(x, edge_index, W1, b1, W2, b2):
    src = edge_index[0].astype(jnp.int32)
    dst = edge_index[1].astype(jnp.int32)
    npad = EPAD - E
    pi = jnp.arange(npad, dtype=jnp.int32)
    srcp = jnp.concatenate([src, pi % PAD_ROWS])
    dstp = jnp.concatenate([dst, N + (pi % PAD_ROWS)])
    srcp_a = srcp.reshape(EPAD // AWIN, AWIN)
    dstp_a = dstp.reshape(EPAD // AWIN, AWIN)
    dstp_d = dstp.reshape(EPAD // WIN, WIN)
    zeros_init = jnp.zeros((TROW_LAST, F), jnp.float32)

    degA, degB = _deg_call(dstp_d, zeros_init)
    xw1 = _tc_matmul(x, W1)
    y0, y1 = _tc_scale_split(xw1, degA, degB)
    z0, z1 = _agg_call(y0, y1, srcp_a, dstp_a)
    y0, y1 = _tc_mid(z0, z1, degA, degB,
                     b1.reshape(1, H), W2)
    z0, z1 = _agg_call(y0, y1, srcp_a, dstp_a)
    return _tc_final(z0, z1, degA, degB, b2.reshape(1, H))


# speedup vs baseline: 1.2089x; 1.2089x over previous
import dataclasses
import functools

import jax
import jax.numpy as jnp
from jax import lax
from jax.experimental import pallas as pl
from jax.experimental.pallas import tpu as pltpu
from jax.experimental.pallas import tpu_sc as plsc

N = 10000
E = 320000
F = 128
H = 256
NC = 2
NS = 16
WIN = 128
W_AGG = 160
EPP = W_AGG * WIN
EPAD = EPP * NS
W_DEG = 80
DPP = W_DEG * WIN
PAD_ROWS = 16
NPT = N + PAD_ROWS
ROW_CHUNK = 624
ROW_LAST = N - 15 * ROW_CHUNK
TROW_LAST = NPT - 15 * ROW_CHUNK
RB = 2000

_mesh = plsc.VectorSubcoreMesh(core_axis_name="c", subcore_axis_name="s")


def _chunked_copy(copy_fn, sid, last_size):
    base = sid * ROW_CHUNK

    @pl.when(sid < 15)
    def _():
        copy_fn(base, ROW_CHUNK)

    @pl.when(sid == 15)
    def _():
        copy_fn(base, last_size)


DEG_E = EPAD // (NC * NS)
HIST_N = 10240
HCOL = HIST_N // NS
HLAST = N - 15 * HCOL


def _deg_body(dstp_hbm, degA_hbm, degB_hbm, idx_v, hist_v, stage_v, acc_v,
              hists_sh):
    cid = lax.axis_index("c")
    sid = lax.axis_index("s")
    wid = cid * NS + sid

    zeros = jnp.zeros((16,), jnp.float32)

    @pl.loop(0, HIST_N, step=16)
    def _(i):
        hist_v[pl.ds(i, 16)] = zeros

    pltpu.sync_copy(dstp_hbm.at[pl.ds(wid * DEG_E, DEG_E)], idx_v)

    ones = jnp.full((16,), 1.0, dtype=jnp.float32)

    @pl.loop(0, DEG_E, step=16)
    def _(i):
        iv = idx_v[pl.ds(i, 16)]
        plsc.addupdate_scatter(hist_v, [iv], ones)

    pltpu.sync_copy(hist_v, hists_sh.at[sid])
    plsc.subcore_barrier()

    def reduce_out(deg_hbm):
        colbase = sid * HCOL
        pltpu.sync_copy(hists_sh.at[:, pl.ds(colbase, HCOL)], stage_v)

        @pl.loop(0, HCOL, step=16)
        def _(off):
            v = stage_v[0, pl.ds(off, 16)]
            for r in range(1, NS):
                v = v + stage_v[r, pl.ds(off, 16)]
            acc_v[pl.ds(off, 16)] = v

        @pl.when(sid < 15)
        def _():
            pltpu.sync_copy(acc_v, deg_hbm.at[pl.ds(colbase, HCOL)])

        @pl.when(sid == 15)
        def _():
            pltpu.sync_copy(acc_v.at[pl.ds(0, HLAST)],
                            deg_hbm.at[pl.ds(colbase, HLAST)])

    @pl.when(cid == 0)
    def _():
        reduce_out(degA_hbm)

    @pl.when(cid == 1)
    def _():
        reduce_out(degB_hbm)


_deg_cp = pltpu.CompilerParams()
if "needs_layout_passes" in pltpu.CompilerParams.__dataclass_fields__:
    _deg_cp = dataclasses.replace(_deg_cp, needs_layout_passes=False)

_deg_call = functools.partial(
    pl.kernel,
    out_type=[jax.ShapeDtypeStruct((N,), jnp.float32),
              jax.ShapeDtypeStruct((N,), jnp.float32)],
    mesh=_mesh,
    compiler_params=_deg_cp,
    scratch_types=[
        pltpu.VMEM((DEG_E,), jnp.int32),
        pltpu.VMEM((HIST_N,), jnp.float32),
        pltpu.VMEM((NS, HCOL), jnp.float32),
        pltpu.VMEM((HCOL,), jnp.float32),
        pltpu.VMEM_SHARED((NS, HIST_N), jnp.float32),
    ],
)(_deg_body)


CH = 16
NCH = W_AGG // CH


def _agg_body(y0_hbm, y1_hbm, srcp_hbm, dstp_hbm, z0_hbm, z1_hbm,
              cs0, cs1, cd0, cd1, rows0, rows1,
              semc0, semc1, semg0, semg1, zsh):
    cid = lax.axis_index("c")
    sid = lax.axis_index("s")
    cs = (cs0, cs1)
    cd = (cd0, cd1)
    semc = (semc0, semc1)
    rows = (rows0, rows1)
    semg = (semg0, semg1)

    def run(y_hbm, z_hbm):
        def seed(base, size):
            pltpu.sync_copy(y_hbm.at[pl.ds(base, size)],
                            zsh.at[pl.ds(base, size)])

        _chunked_copy(seed, sid, ROW_LAST)
        plsc.subcore_barrier()

        def start_chunk(c, s):
            base = sid * W_AGG + c * CH
            pltpu.async_copy(srcp_hbm.at[pl.ds(base, CH)], cs[s], semc[s])
            pltpu.async_copy(dstp_hbm.at[pl.ds(base, CH)], cd[s], semc[s])

        def wait_chunk(s):
            pltpu.make_async_copy(srcp_hbm.at[pl.ds(0, CH)], cs[s],
                                  semc[s]).wait()
            pltpu.make_async_copy(dstp_hbm.at[pl.ds(0, CH)], cd[s],
                                  semc[s]).wait()

        def start_gather(s, k, b):
            pltpu.async_copy(y_hbm.at[cs[s].at[k]], rows[b], semg[b])

        def wait_gather(b):
            pltpu.make_async_copy(y_hbm.at[cs[0].at[0]], rows[b],
                                  semg[b]).wait()

        start_chunk(0, 0)
        start_chunk(1, 1)

        @pl.loop(0, NCH, step=2)
        def _(c):
            for s in range(2):
                wait_chunk(s)
                start_gather(s, 0, 0)
                start_gather(s, 1, 1)
                for k in range(CH):
                    b = k % 2
                    wait_gather(b)
                    pltpu.sync_copy(rows[b], zsh.at[cd[s].at[k]], add=True)
                    if k + 2 < CH:
                        start_gather(s, k + 2, b)

                @pl.when(c + 2 + s < NCH)
                def _():
                    start_chunk(c + 2 + s, s)

        plsc.subcore_barrier()

        def out(base, size):
            pltpu.sync_copy(zsh.at[pl.ds(base, size)],
                            z_hbm.at[pl.ds(base, size)])

        _chunked_copy(out, sid, ROW_LAST)

    @pl.when(cid == 0)
    def _():
        run(y0_hbm, z0_hbm)

    @pl.when(cid == 1)
    def _():
        run(y1_hbm, z1_hbm)


_agg_call = functools.partial(
    pl.kernel,
    out_type=[jax.ShapeDtypeStruct((N, F), jnp.float32),
              jax.ShapeDtypeStruct((N, F), jnp.float32)],
    mesh=_mesh,
    scratch_types=[
        pltpu.VMEM((CH, WIN), jnp.int32),
        pltpu.VMEM((CH, WIN), jnp.int32),
        pltpu.VMEM((CH, WIN), jnp.int32),
        pltpu.VMEM((CH, WIN), jnp.int32),
        pltpu.VMEM((WIN, F), jnp.float32),
        pltpu.VMEM((WIN, F), jnp.float32),
        pltpu.SemaphoreType.DMA,
        pltpu.SemaphoreType.DMA,
        pltpu.SemaphoreType.DMA,
        pltpu.SemaphoreType.DMA,
        pltpu.VMEM_SHARED((NPT, F), jnp.float32),
    ],
)(_agg_body)


def _mm_body(x_ref, w_ref, o_ref):
    o_ref[...] = jnp.dot(x_ref[...], w_ref[...],
                         preferred_element_type=jnp.float32,
                         precision=lax.Precision.HIGHEST)


def _tc_matmul(x, w):
    m, k = x.shape
    _, n = w.shape
    return pl.pallas_call(
        _mm_body,
        grid=(m // RB,),
        in_specs=[pl.BlockSpec((RB, k), lambda i: (i, 0)),
                  pl.BlockSpec((k, n), lambda i: (0, 0))],
        out_specs=pl.BlockSpec((RB, n), lambda i: (i, 0)),
        out_shape=jax.ShapeDtypeStruct((m, n), jnp.float32),
    )(x, w)


def _scale_body(xw_ref, degA_ref, degB_ref, y0_ref, y1_ref):
    deg = degA_ref[...] + degB_ref[...]
    dinv = lax.rsqrt(deg + 1.0)
    y = xw_ref[...] * dinv
    y0_ref[...] = y[:, :F]
    y1_ref[...] = y[:, F:]


def _tc_scale_split(xw, degA, degB):
    return pl.pallas_call(
        _scale_body,
        grid=(N // RB,),
        in_specs=[pl.BlockSpec((RB, H), lambda i: (i, 0)),
                  pl.BlockSpec((RB, 1), lambda i: (i, 0)),
                  pl.BlockSpec((RB, 1), lambda i: (i, 0))],
        out_specs=[pl.BlockSpec((RB, F), lambda i: (i, 0)),
                   pl.BlockSpec((RB, F), lambda i: (i, 0))],
        out_shape=[jax.ShapeDtypeStruct((N, F), jnp.float32),
                   jax.ShapeDtypeStruct((N, F), jnp.float32)],
    )(xw, degA, degB)


def _mid_body(z0_ref, z1_ref, degA_ref, degB_ref, b_ref, w_ref,
              y0_ref, y1_ref):
    deg = degA_ref[...] + degB_ref[...]
    dinv = lax.rsqrt(deg + 1.0)
    h = jnp.concatenate([z0_ref[...], z1_ref[...]], axis=1)
    h = jnp.maximum(h * dinv + b_ref[...], 0.0)
    xw = jnp.dot(h, w_ref[...],
                 preferred_element_type=jnp.float32,
                 precision=lax.Precision.HIGHEST)
    y = xw * dinv
    y0_ref[...] = y[:, :F]
    y1_ref[...] = y[:, F:]


def _tc_mid(z0, z1, degA, degB, b, w):
    return pl.pallas_call(
        _mid_body,
        grid=(N // RB,),
        in_specs=[pl.BlockSpec((RB, F), lambda i: (i, 0)),
                  pl.BlockSpec((RB, F), lambda i: (i, 0)),
                  pl.BlockSpec((RB, 1), lambda i: (i, 0)),
                  pl.BlockSpec((RB, 1), lambda i: (i, 0)),
                  pl.BlockSpec((1, H), lambda i: (0, 0)),
                  pl.BlockSpec((H, H), lambda i: (0, 0))],
        out_specs=[pl.BlockSpec((RB, F), lambda i: (i, 0)),
                   pl.BlockSpec((RB, F), lambda i: (i, 0))],
        out_shape=[jax.ShapeDtypeStruct((N, F), jnp.float32),
                   jax.ShapeDtypeStruct((N, F), jnp.float32)],
    )(z0, z1, degA, degB, b, w)


def _fin_body(z0_ref, z1_ref, degA_ref, degB_ref, b_ref, o_ref):
    deg = degA_ref[...] + degB_ref[...]
    dinv = lax.rsqrt(deg + 1.0)
    h = jnp.concatenate([z0_ref[...], z1_ref[...]], axis=1)
    o_ref[...] = jnp.maximum(h * dinv + b_ref[...], 0.0)


def _tc_final(z0, z1, degA, degB, b):
    return pl.pallas_call(
        _fin_body,
        grid=(N // RB,),
        in_specs=[pl.BlockSpec((RB, F), lambda i: (i, 0)),
                  pl.BlockSpec((RB, F), lambda i: (i, 0)),
                  pl.BlockSpec((RB, 1), lambda i: (i, 0)),
                  pl.BlockSpec((RB, 1), lambda i: (i, 0)),
                  pl.BlockSpec((1, H), lambda i: (0, 0))],
        out_specs=pl.BlockSpec((RB, H), lambda i: (i, 0)),
        out_shape=jax.ShapeDtypeStruct((N, H), jnp.float32),
    )(z0, z1, degA, degB, b)


def kernel(x, edge_index, W1, b1, W2, b2):
    src = edge_index[0].astype(jnp.int32)
    dst = edge_index[1].astype(jnp.int32)
    npad = EPAD - E
    pi = jnp.arange(npad, dtype=jnp.int32)
    srcp = jnp.concatenate([src, pi % PAD_ROWS])
    dstp = jnp.concatenate([dst, N + (pi % PAD_ROWS)])
    srcp_a = srcp.reshape(EPAD // WIN, WIN)
    dstp_a = dstp.reshape(EPAD // WIN, WIN)

    degA, degB = _deg_call(dstp)
    degA = degA.reshape(N, 1)
    degB = degB.reshape(N, 1)
    xw1 = _tc_matmul(x, W1)
    y0, y1 = _tc_scale_split(xw1, degA, degB)
    z0, z1 = _agg_call(y0, y1, srcp_a, dstp_a)
    y0, y1 = _tc_mid(z0, z1, degA, degB,
                     b1.reshape(1, H), W2)
    z0, z1 = _agg_call(y0, y1, srcp_a, dstp_a)
    return _tc_final(z0, z1, degA, degB, b2.reshape(1, H))


# speedup vs baseline: 1.2255x; 1.0137x over previous
import dataclasses
import functools

import jax
import jax.numpy as jnp
import numpy as np
from jax import lax
from jax.experimental import pallas as pl
from jax.experimental.pallas import tpu as pltpu
from jax.experimental.pallas import tpu_sc as plsc

N = 10000
E = 320000
F = 128
H = 256
NC = 2
NS = 16
WIN = 128
W_AGG = 160
EPP = W_AGG * WIN
EPAD = EPP * NS
W_DEG = 80
DPP = W_DEG * WIN
PAD_ROWS = 16
NPT = N + PAD_ROWS
ROW_CHUNK = 624
ROW_LAST = N - 15 * ROW_CHUNK
TROW_LAST = NPT - 15 * ROW_CHUNK
RB = 2000

_mesh = plsc.VectorSubcoreMesh(core_axis_name="c", subcore_axis_name="s")


def _chunked_copy(copy_fn, sid, last_size):
    base = sid * ROW_CHUNK

    @pl.when(sid < 15)
    def _():
        copy_fn(base, ROW_CHUNK)

    @pl.when(sid == 15)
    def _():
        copy_fn(base, last_size)


DEG_E = EPAD // (NC * NS)
HIST_N = 10240
HCOL = HIST_N // NS
HLAST = N - 15 * HCOL


def _deg_body(dstp_hbm, degA_hbm, degB_hbm, idx_v, hist_v, stage_v, acc_v,
              hists_sh):
    cid = lax.axis_index("c")
    sid = lax.axis_index("s")
    wid = cid * NS + sid

    zeros = jnp.zeros((16,), jnp.float32)

    @pl.loop(0, HIST_N, step=16)
    def _(i):
        hist_v[pl.ds(i, 16)] = zeros

    pltpu.sync_copy(dstp_hbm.at[pl.ds(wid * DEG_E, DEG_E)], idx_v)

    ones = jnp.full((16,), 1.0, dtype=jnp.float32)

    @pl.loop(0, DEG_E, step=16)
    def _(i):
        iv = idx_v[pl.ds(i, 16)]
        plsc.addupdate_scatter(hist_v, [iv], ones)

    pltpu.sync_copy(hist_v, hists_sh.at[sid])
    plsc.subcore_barrier()

    def reduce_out(deg_hbm):
        colbase = sid * HCOL
        pltpu.sync_copy(hists_sh.at[:, pl.ds(colbase, HCOL)], stage_v)

        @pl.loop(0, HCOL, step=16)
        def _(off):
            v = stage_v[0, pl.ds(off, 16)]
            for r in range(1, NS):
                v = v + stage_v[r, pl.ds(off, 16)]
            acc_v[pl.ds(off, 16)] = v

        @pl.when(sid < 15)
        def _():
            pltpu.sync_copy(acc_v, deg_hbm.at[pl.ds(colbase, HCOL)])

        @pl.when(sid == 15)
        def _():
            pltpu.sync_copy(acc_v.at[pl.ds(0, HLAST)],
                            deg_hbm.at[pl.ds(colbase, HLAST)])

    @pl.when(cid == 0)
    def _():
        reduce_out(degA_hbm)

    @pl.when(cid == 1)
    def _():
        reduce_out(degB_hbm)


_deg_cp = pltpu.CompilerParams()
if "needs_layout_passes" in pltpu.CompilerParams.__dataclass_fields__:
    _deg_cp = dataclasses.replace(_deg_cp, needs_layout_passes=False)

_deg_call = functools.partial(
    pl.kernel,
    out_type=[jax.ShapeDtypeStruct((N,), jnp.float32),
              jax.ShapeDtypeStruct((N,), jnp.float32)],
    mesh=_mesh,
    compiler_params=_deg_cp,
    scratch_types=[
        pltpu.VMEM((DEG_E,), jnp.int32),
        pltpu.VMEM((HIST_N,), jnp.float32),
        pltpu.VMEM((NS, HCOL), jnp.float32),
        pltpu.VMEM((HCOL,), jnp.float32),
        pltpu.VMEM_SHARED((NS, HIST_N), jnp.float32),
    ],
)(_deg_body)


CH = 16
NCH = W_AGG // CH


def _agg_body(y0_hbm, y1_hbm, srcp_hbm, dstp_hbm, z0_hbm, z1_hbm,
              cs0, cs1, cd0, cd1, rows0, rows1,
              semc0, semc1, semg0, semg1, zsh):
    cid = lax.axis_index("c")
    sid = lax.axis_index("s")
    cs = (cs0, cs1)
    cd = (cd0, cd1)
    semc = (semc0, semc1)
    rows = (rows0, rows1)
    semg = (semg0, semg1)

    def run(y_hbm, z_hbm):
        def seed(base, size):
            pltpu.sync_copy(y_hbm.at[pl.ds(base, size)],
                            zsh.at[pl.ds(base, size)])

        _chunked_copy(seed, sid, ROW_LAST)
        plsc.subcore_barrier()

        def start_chunk(c, s):
            base = sid * W_AGG + c * CH
            pltpu.async_copy(srcp_hbm.at[pl.ds(base, CH)], cs[s], semc[s])
            pltpu.async_copy(dstp_hbm.at[pl.ds(base, CH)], cd[s], semc[s])

        def wait_chunk(s):
            pltpu.make_async_copy(srcp_hbm.at[pl.ds(0, CH)], cs[s],
                                  semc[s]).wait()
            pltpu.make_async_copy(dstp_hbm.at[pl.ds(0, CH)], cd[s],
                                  semc[s]).wait()

        def start_gather(s, k, b):
            pltpu.async_copy(y_hbm.at[cs[s].at[k]], rows[b], semg[b])

        def wait_gather(b):
            pltpu.make_async_copy(y_hbm.at[cs[0].at[0]], rows[b],
                                  semg[b]).wait()

        start_chunk(0, 0)
        start_chunk(1, 1)

        @pl.loop(0, NCH, step=2)
        def _(c):
            for s in range(2):
                wait_chunk(s)
                start_gather(s, 0, 0)
                start_gather(s, 1, 1)
                for k in range(CH):
                    b = k % 2
                    wait_gather(b)
                    pltpu.sync_copy(rows[b], zsh.at[cd[s].at[k]], add=True)
                    if k + 2 < CH:
                        start_gather(s, k + 2, b)

                @pl.when(c + 2 + s < NCH)
                def _():
                    start_chunk(c + 2 + s, s)

        plsc.subcore_barrier()

        def out(base, size):
            pltpu.sync_copy(zsh.at[pl.ds(base, size)],
                            z_hbm.at[pl.ds(base, size)])

        _chunked_copy(out, sid, ROW_LAST)

    @pl.when(cid == 0)
    def _():
        run(y0_hbm, z0_hbm)

    @pl.when(cid == 1)
    def _():
        run(y1_hbm, z1_hbm)


_agg_call = functools.partial(
    pl.kernel,
    out_type=[jax.ShapeDtypeStruct((N, F), jnp.float32),
              jax.ShapeDtypeStruct((N, F), jnp.float32)],
    mesh=_mesh,
    scratch_types=[
        pltpu.VMEM((CH, WIN), jnp.int32),
        pltpu.VMEM((CH, WIN), jnp.int32),
        pltpu.VMEM((CH, WIN), jnp.int32),
        pltpu.VMEM((CH, WIN), jnp.int32),
        pltpu.VMEM((WIN, F), jnp.float32),
        pltpu.VMEM((WIN, F), jnp.float32),
        pltpu.SemaphoreType.DMA,
        pltpu.SemaphoreType.DMA,
        pltpu.SemaphoreType.DMA,
        pltpu.SemaphoreType.DMA,
        pltpu.VMEM_SHARED((NPT, F), jnp.float32),
    ],
)(_agg_body)


def _mm_body(x_ref, w_ref, o_ref):
    o_ref[...] = jnp.dot(x_ref[...], w_ref[...],
                         preferred_element_type=jnp.float32,
                         precision=lax.Precision.HIGHEST)


def _tc_matmul(x, w):
    m, k = x.shape
    _, n = w.shape
    return pl.pallas_call(
        _mm_body,
        grid=(m // RB,),
        in_specs=[pl.BlockSpec((RB, k), lambda i: (i, 0)),
                  pl.BlockSpec((k, n), lambda i: (0, 0))],
        out_specs=pl.BlockSpec((RB, n), lambda i: (i, 0)),
        out_shape=jax.ShapeDtypeStruct((m, n), jnp.float32),
    )(x, w)


def _scale_body(xw_ref, deg_ref, y0_ref, y1_ref):
    dinv = lax.rsqrt(deg_ref[...] + 1.0)
    y = xw_ref[...] * dinv
    y0_ref[...] = y[:, :F]
    y1_ref[...] = y[:, F:]


def _tc_scale_split(xw, deg):
    return pl.pallas_call(
        _scale_body,
        grid=(N // RB,),
        in_specs=[pl.BlockSpec((RB, H), lambda i: (i, 0)),
                  pl.BlockSpec((RB, 1), lambda i: (i, 0))],
        out_specs=[pl.BlockSpec((RB, F), lambda i: (i, 0)),
                   pl.BlockSpec((RB, F), lambda i: (i, 0))],
        out_shape=[jax.ShapeDtypeStruct((N, F), jnp.float32),
                   jax.ShapeDtypeStruct((N, F), jnp.float32)],
    )(xw, deg)


def _mid_body(z0_ref, z1_ref, deg_ref, b_ref, w_ref, y0_ref, y1_ref):
    dinv = lax.rsqrt(deg_ref[...] + 1.0)
    h = jnp.concatenate([z0_ref[...], z1_ref[...]], axis=1)
    h = jnp.maximum(h * dinv + b_ref[...], 0.0)
    xw = jnp.dot(h, w_ref[...],
                 preferred_element_type=jnp.float32,
                 precision=lax.Precision.HIGHEST)
    y = xw * dinv
    y0_ref[...] = y[:, :F]
    y1_ref[...] = y[:, F:]


def _tc_mid(z0, z1, deg, b, w):
    return pl.pallas_call(
        _mid_body,
        grid=(N // RB,),
        in_specs=[pl.BlockSpec((RB, F), lambda i: (i, 0)),
                  pl.BlockSpec((RB, F), lambda i: (i, 0)),
                  pl.BlockSpec((RB, 1), lambda i: (i, 0)),
                  pl.BlockSpec((1, H), lambda i: (0, 0)),
                  pl.BlockSpec((H, H), lambda i: (0, 0))],
        out_specs=[pl.BlockSpec((RB, F), lambda i: (i, 0)),
                   pl.BlockSpec((RB, F), lambda i: (i, 0))],
        out_shape=[jax.ShapeDtypeStruct((N, F), jnp.float32),
                   jax.ShapeDtypeStruct((N, F), jnp.float32)],
    )(z0, z1, deg, b, w)


def _fin_body(z0_ref, z1_ref, deg_ref, b_ref, o_ref):
    dinv = lax.rsqrt(deg_ref[...] + 1.0)
    h = jnp.concatenate([z0_ref[...], z1_ref[...]], axis=1)
    o_ref[...] = jnp.maximum(h * dinv + b_ref[...], 0.0)


def _tc_final(z0, z1, deg, b):
    return pl.pallas_call(
        _fin_body,
        grid=(N // RB,),
        in_specs=[pl.BlockSpec((RB, F), lambda i: (i, 0)),
                  pl.BlockSpec((RB, F), lambda i: (i, 0)),
                  pl.BlockSpec((RB, 1), lambda i: (i, 0)),
                  pl.BlockSpec((1, H), lambda i: (0, 0))],
        out_specs=pl.BlockSpec((RB, H), lambda i: (i, 0)),
        out_shape=jax.ShapeDtypeStruct((N, H), jnp.float32),
    )(z0, z1, deg, b)


_PAD_SRC = np.arange(EPAD - E, dtype=np.int32) % PAD_ROWS
_PAD_DST = (N + (np.arange(EPAD - E, dtype=np.int32) % PAD_ROWS)).astype(np.int32)


def kernel(x, edge_index, W1, b1, W2, b2):
    src = edge_index[0].astype(jnp.int32)
    dst = edge_index[1].astype(jnp.int32)
    srcp_a = jnp.concatenate([src, _PAD_SRC]).reshape(EPAD // WIN, WIN)
    dstp_a = jnp.concatenate([dst, _PAD_DST]).reshape(EPAD // WIN, WIN)

    degA, degB = _deg_call(dstp_a.reshape(EPAD))
    deg = (degA + degB).reshape(N, 1)
    xw1 = _tc_matmul(x, W1)
    y0, y1 = _tc_scale_split(xw1, deg)
    z0, z1 = _agg_call(y0, y1, srcp_a, dstp_a)
    y0, y1 = _tc_mid(z0, z1, deg, b1.reshape(1, H), W2)
    z0, z1 = _agg_call(y0, y1, srcp_a, dstp_a)
    return _tc_final(z0, z1, deg, b2.reshape(1, H))
